# Initial kernel scaffold; baseline (speedup 1.0000x reference)
import jax
import jax.numpy as jnp
from jax.experimental import pallas as pl


def kernel(x, router_DE, w13, w2, shared_w13, shared_w2):
    raise NotImplementedError("write your pallas kernel here")


# speedup vs baseline: 1.4303x; 1.4303x over previous
import functools

import jax
import jax.numpy as jnp
from jax.experimental import pallas as pl
from jax.experimental.pallas import tpu as pltpu


def _router_body(E, x_ref, r_ref, comb_ref):
    logits = jnp.dot(x_ref[...], r_ref[...], preferred_element_type=jnp.float32)
    iota = jax.lax.broadcasted_iota(jnp.int32, logits.shape, 1)
    m1 = jnp.max(logits, axis=-1, keepdims=True)
    i1 = jnp.min(jnp.where(logits == m1, iota, E), axis=-1, keepdims=True)
    masked = jnp.where(iota == i1, jnp.float32(-1e30), logits)
    m2 = jnp.max(masked, axis=-1, keepdims=True)
    i2 = jnp.min(jnp.where(masked == m2, iota, E), axis=-1, keepdims=True)
    e2 = jnp.exp(m2 - m1)
    w1 = 1.0 / (1.0 + e2)
    w2 = e2 / (1.0 + e2)
    comb_ref[...] = (jnp.where(iota == i1, w1, 0.0)
                     + jnp.where(iota == i2, w2, 0.0))


def _moe_body(E, x_ref, w13_ref, sh13_ref, w2_ref, sh2_ref, comb_ref, out_ref):
    f = pl.program_id(0)
    e = pl.program_id(1)

    @pl.when((f == 0) & (e == 0))
    def _init():
        out_ref[...] = jnp.zeros_like(out_ref)

    is_shared = e == E
    w13b = jnp.where(is_shared, sh13_ref[0], w13_ref[0]).astype(jnp.bfloat16)
    xb = x_ref[...]
    h1 = jax.lax.dot_general(xb, w13b[0], (((1,), (1,)), ((), ())),
                             preferred_element_type=jnp.float32)
    h3 = jax.lax.dot_general(xb, w13b[1], (((1,), (1,)), ((), ())),
                             preferred_element_type=jnp.float32)
    act = (h1 * jax.nn.sigmoid(h1) * h3).astype(jnp.bfloat16)
    w2b = jnp.where(is_shared, sh2_ref[0], w2_ref[0]).astype(jnp.bfloat16)
    contrib = jax.lax.dot_general(act, w2b, (((1,), (1,)), ((), ())),
                                  preferred_element_type=jnp.float32)
    comb = comb_ref[...]
    lane = jax.lax.broadcasted_iota(jnp.int32, comb.shape, 1)
    col = jnp.sum(jnp.where(lane == jnp.minimum(e, E - 1), comb, 0.0),
                  axis=1, keepdims=True)
    scale = jnp.where(is_shared, 1.0, col)
    out_ref[...] += scale * contrib


def kernel(x, router_DE, w13, w2, shared_w13, shared_w2):
    T, D = x.shape
    E, twoF, _ = w13.shape
    F = twoF // 2

    comb = pl.pallas_call(
        functools.partial(_router_body, E),
        out_shape=jax.ShapeDtypeStruct((T, E), jnp.float32),
    )(x, router_DE)

    xbf = x.astype(jnp.bfloat16)
    w13r = w13.reshape(E, 2, F, D)
    sh13r = shared_w13.reshape(1, 2, F, D)
    sh2r = shared_w2.reshape(1, D, F)
    bf = min(512, F)
    nf = F // bf

    out = pl.pallas_call(
        functools.partial(_moe_body, E),
        grid=(nf, E + 1),
        in_specs=[
            pl.BlockSpec((T, D), lambda f, e: (0, 0)),
            pl.BlockSpec((1, 2, bf, D), lambda f, e: (jnp.minimum(e, E - 1), 0, f, 0)),
            pl.BlockSpec((1, 2, bf, D), lambda f, e: (0, 0, f, 0)),
            pl.BlockSpec((1, D, bf), lambda f, e: (jnp.minimum(e, E - 1), 0, f)),
            pl.BlockSpec((1, D, bf), lambda f, e: (0, 0, f)),
            pl.BlockSpec((T, E), lambda f, e: (0, 0)),
        ],
        out_specs=pl.BlockSpec((T, D), lambda f, e: (0, 0)),
        out_shape=jax.ShapeDtypeStruct((T, D), jnp.float32),
        compiler_params=pltpu.CompilerParams(
            dimension_semantics=("arbitrary", "arbitrary")),
    )(xbf, w13r, sh13r, w2, sh2r, comb)
    return out
